# baseline (speedup 1.0000x reference)
import jax
import jax.numpy as jnp
from jax import lax
from jax.experimental import pallas as pl
from jax.experimental.pallas import tpu as pltpu
from jax.experimental.pallas import tpu_sc as plsc

_D = 16
_B = 16384
_NC = 2
_NS = 16
_NW = _NC * _NS
_BPW = _B // _NW
_IPC = 64
_NG = _BPW // _IPC
_RP = 2 * _D
_T01 = 41 * 37
_TROWS = _T01 + 31 * 23
_RPW = 140


def _body(ids_hbm, w0_hbm, w1_hbm, w2_hbm, w3_hbm, out_hbm,
          ids_v, wall_v, stage_v, idx0_v, idx1_v, r0_v, r1_v, tr_v,
          shared, gsem, osem):
    wid = lax.axis_index("s") * _NC + lax.axis_index("c")
    sid = lax.axis_index("s")
    base = wid * _BPW
    pltpu.sync_copy(w0_hbm, wall_v.at[pl.ds(0, 41)])
    pltpu.sync_copy(w1_hbm, wall_v.at[pl.ds(41, 37)])
    pltpu.sync_copy(w2_hbm, wall_v.at[pl.ds(78, 31)])
    pltpu.sync_copy(w3_hbm, wall_v.at[pl.ds(109, 23)])
    pltpu.sync_copy(ids_hbm.at[pl.ds(base, _BPW)], ids_v)

    lo = sid * _RPW

    @pl.loop(0, _RPW)
    def _build_rows(k):
        r = lo + k

        @pl.when(r < _T01)
        def _():
            i0 = lax.div(r, 37)
            i1 = r - i0 * 37
            stage_v[k, pl.ds(0, _D)] = wall_v[i0]
            stage_v[k, pl.ds(_D, _D)] = wall_v[41 + i1]

        @pl.when(r >= _T01)
        def _():
            s = r - _T01
            i2 = lax.div(s, 23)
            i3 = s - i2 * 23
            stage_v[k, pl.ds(0, _D)] = wall_v[78 + i2]
            stage_v[k, pl.ds(_D, _D)] = wall_v[109 + i3]

    pltpu.sync_copy(stage_v, shared.at[pl.ds(lo, _RPW)])
    plsc.subcore_barrier()

    lane = lax.iota(jnp.int32, 16)

    def _mod(v, vf, p, recip):
        q = (vf * jnp.float32(recip)).astype(jnp.int32)
        r = v - q * p
        return jnp.where(r >= p, r - p, r)

    @pl.loop(0, _NG)
    def _build_and_gather(g):
        for c in range(_IPC // 16):
            ids = ids_v[pl.ds(g * _IPC + c * 16, 16)]
            idsf = ids.astype(jnp.float32)
            i01 = (_mod(ids, idsf, 41, 1.0 / 41.0) * 37
                   + _mod(ids, idsf, 37, 1.0 / 37.0))
            i23 = (_mod(ids, idsf, 31, 1.0 / 31.0) * 23
                   + _mod(ids, idsf, 23, 1.0 / 23.0) + _T01)
            idx0_v[pl.ds(g * _IPC + c * 16, 16)] = i01
            idx1_v[pl.ds(g * _IPC + c * 16, 16)] = i23
        pltpu.async_copy(
            shared.at[idx0_v.at[pl.ds(g * _IPC, _IPC)]],
            r0_v.at[pl.ds(g * _IPC, _IPC)],
            gsem.at[g],
        )
        pltpu.async_copy(
            shared.at[idx1_v.at[pl.ds(g * _IPC, _IPC)]],
            r1_v.at[pl.ds(g * _IPC, _IPC)],
            gsem.at[g],
        )

    @pl.loop(0, _NG)
    def _drain_transpose_store(g):
        pltpu.make_async_copy(
            shared.at[idx0_v.at[pl.ds(g * _IPC, _IPC)]],
            r0_v.at[pl.ds(g * _IPC, _IPC)],
            gsem.at[g],
        ).wait()
        pltpu.make_async_copy(
            shared.at[idx1_v.at[pl.ds(g * _IPC, _IPC)]],
            r1_v.at[pl.ds(g * _IPC, _IPC)],
            gsem.at[g],
        ).wait()
        for c in range(_IPC // 16):
            i0 = g * _IPC + c * 16
            rvec = lane + i0
            for d in range(2 * _D):
                cvec = (lane + d) & (2 * _D - 1)
                v0 = plsc.load_gather(r0_v, [rvec, cvec])
                plsc.store_scatter(tr_v, [cvec, rvec], v0)
                v1 = plsc.load_gather(r1_v, [rvec, cvec])
                plsc.store_scatter(tr_v, [cvec + 2 * _D, rvec], v1)

    pltpu.sync_copy(tr_v, out_hbm.at[:, pl.ds(base, _BPW)])


def kernel(user_ids, W0, W1, W2, W3):
    ids = user_ids.astype(jnp.int32)
    mesh = plsc.VectorSubcoreMesh(core_axis_name="c", subcore_axis_name="s")
    out = pl.kernel(
        _body,
        mesh=mesh,
        compiler_params=pltpu.CompilerParams(
            use_tc_tiling_on_sc=False, needs_layout_passes=False),
        out_type=jax.ShapeDtypeStruct((4 * _D, _B), jnp.float32),
        scratch_types=[
            pltpu.VMEM((_BPW,), jnp.int32),
            pltpu.VMEM((132, _D), jnp.float32),
            pltpu.VMEM((_RPW, _RP), jnp.float32),
            pltpu.VMEM((_BPW,), jnp.int32),
            pltpu.VMEM((_BPW,), jnp.int32),
            pltpu.VMEM((_BPW, _RP), jnp.float32),
            pltpu.VMEM((_BPW, _RP), jnp.float32),
            pltpu.VMEM((4 * _D, _BPW), jnp.float32),
            pltpu.VMEM_SHARED((_NS * _RPW, _RP), jnp.float32),
            pltpu.SemaphoreType.DMA((_NG,)),
            pltpu.SemaphoreType.DMA,
        ],
    )(ids, W0, W1, W2, W3)
    return out.T


# speedup vs baseline: 1.1046x; 1.1046x over previous
import jax
import jax.numpy as jnp
from jax import lax
from jax.experimental import pallas as pl
from jax.experimental.pallas import tpu as pltpu
from jax.experimental.pallas import tpu_sc as plsc

_D = 16
_B = 16384
_NC = 2
_NS = 16
_NW = _NC * _NS
_BPW = _B // _NW
_IPC = 64
_NG = _BPW // _IPC
_RP = 2 * _D
_T01 = 41 * 37


def _body(ids_hbm, tcat_hbm, out_hbm,
          ids_v, idx0_v, idx1_v, r0_v, r1_v, tr_v, gsem, osem):
    wid = lax.axis_index("s") * _NC + lax.axis_index("c")
    base = wid * _BPW
    pltpu.sync_copy(ids_hbm.at[pl.ds(base, _BPW)], ids_v)
    lane = lax.iota(jnp.int32, 16)

    def _mod(v, vf, p, recip):
        q = (vf * jnp.float32(recip)).astype(jnp.int32)
        r = v - q * p
        return jnp.where(r >= p, r - p, r)

    @pl.loop(0, _NG)
    def _build_and_gather(g):
        for c in range(_IPC // 16):
            ids = ids_v[pl.ds(g * _IPC + c * 16, 16)]
            idsf = ids.astype(jnp.float32)
            i01 = (_mod(ids, idsf, 41, 1.0 / 41.0) * 37
                   + _mod(ids, idsf, 37, 1.0 / 37.0))
            i23 = (_mod(ids, idsf, 31, 1.0 / 31.0) * 23
                   + _mod(ids, idsf, 23, 1.0 / 23.0) + _T01)
            idx0_v[pl.ds(g * _IPC + c * 16, 16)] = i01
            idx1_v[pl.ds(g * _IPC + c * 16, 16)] = i23
        pltpu.async_copy(
            tcat_hbm.at[idx0_v.at[pl.ds(g * _IPC, _IPC)]],
            r0_v.at[pl.ds(g * _IPC, _IPC)],
            gsem.at[g],
        )
        pltpu.async_copy(
            tcat_hbm.at[idx1_v.at[pl.ds(g * _IPC, _IPC)]],
            r1_v.at[pl.ds(g * _IPC, _IPC)],
            gsem.at[g],
        )

    @pl.loop(0, _NG)
    def _drain_transpose_store(g):
        pltpu.make_async_copy(
            tcat_hbm.at[idx0_v.at[pl.ds(g * _IPC, _IPC)]],
            r0_v.at[pl.ds(g * _IPC, _IPC)],
            gsem.at[g],
        ).wait()
        pltpu.make_async_copy(
            tcat_hbm.at[idx1_v.at[pl.ds(g * _IPC, _IPC)]],
            r1_v.at[pl.ds(g * _IPC, _IPC)],
            gsem.at[g],
        ).wait()
        for c in range(_IPC // 16):
            i0 = g * _IPC + c * 16
            rvec = lane + i0
            for d in range(2 * _D):
                cvec = (lane + d) & (2 * _D - 1)
                v0 = plsc.load_gather(r0_v, [rvec, cvec])
                plsc.store_scatter(tr_v, [cvec, rvec], v0)
                v1 = plsc.load_gather(r1_v, [rvec, cvec])
                plsc.store_scatter(tr_v, [cvec + 2 * _D, rvec], v1)

    pltpu.sync_copy(tr_v, out_hbm.at[:, pl.ds(base, _BPW)])


def kernel(user_ids, W0, W1, W2, W3):
    t01 = jnp.concatenate(
        [jnp.repeat(W0, 37, axis=0), jnp.tile(W1, (41, 1))], axis=1)
    t23 = jnp.concatenate(
        [jnp.repeat(W2, 23, axis=0), jnp.tile(W3, (31, 1))], axis=1)
    tcat = jnp.concatenate([t01, t23], axis=0)
    ids = user_ids.astype(jnp.int32)
    mesh = plsc.VectorSubcoreMesh(core_axis_name="c", subcore_axis_name="s")
    out = pl.kernel(
        _body,
        mesh=mesh,
        compiler_params=pltpu.CompilerParams(
            use_tc_tiling_on_sc=False, needs_layout_passes=False),
        out_type=jax.ShapeDtypeStruct((4 * _D, _B), jnp.float32),
        scratch_types=[
            pltpu.VMEM((_BPW,), jnp.int32),
            pltpu.VMEM((_BPW,), jnp.int32),
            pltpu.VMEM((_BPW,), jnp.int32),
            pltpu.VMEM((_BPW, _RP), jnp.float32),
            pltpu.VMEM((_BPW, _RP), jnp.float32),
            pltpu.VMEM((4 * _D, _BPW), jnp.float32),
            pltpu.SemaphoreType.DMA((_NG,)),
            pltpu.SemaphoreType.DMA,
        ],
    )(ids, tcat)
    return out.T
